# Initial kernel scaffold; baseline (speedup 1.0000x reference)
import jax
import jax.numpy as jnp
from jax.experimental import pallas as pl


def kernel(x, edge_index, edge_attr, batch, W_node, b_node, W_e1, b_e1, W_e2, b_e2, convs, bns, W_p1, b_p1, W_p2, b_p2):
    raise NotImplementedError("write your pallas kernel here")


# speedup vs baseline: 3.5558x; 3.5558x over previous
import functools

import jax
import jax.numpy as jnp
from jax import lax
from jax.experimental import pallas as pl
from jax.experimental.pallas import tpu as pltpu
from jax.experimental.pallas import tpu_sc as plsc

N = 10000
E = 320000
D = 128
DE = 16
H = 64
P = 64
G = 32

NC = 2
NS = 16
NW = NC * NS
GSZ = 128
NGRP = E // GSZ
CH = 200
NCH = N // CH

_f32 = jnp.float32


def _mp_body(h_hbm, e_hbm, src_hbm, dst_hbm, out_hbm,
             sidx, didx, hbuf, ebuf, zbuf, aggr, sem):
    c = lax.axis_index("c")
    s = lax.axis_index("s")
    wid = s * NC + c

    def _zr(r, _):
        for cc in range(H // 16):
            zbuf[r, pl.ds(cc * 16, 16)] = jnp.zeros((16,), _f32)
        return 0
    lax.fori_loop(0, CH, _zr, 0)
    for k in range((NCH + NS - 1) // NS):
        ch = s + k * NS

        @pl.when(ch < NCH)
        def _():
            pltpu.sync_copy(zbuf, aggr.at[pl.ds(ch * CH, CH), :])
    plsc.subcore_barrier()

    ng = jnp.where(wid < NGRP - (NGRP // NW) * NW, NGRP // NW + 1, NGRP // NW)

    def _group(g, _):
        goff = (g * NW + wid) * GSZ
        pltpu.sync_copy(src_hbm.at[pl.ds(goff, GSZ)], sidx)
        pltpu.sync_copy(dst_hbm.at[pl.ds(goff, GSZ)], didx)
        pltpu.sync_copy(e_hbm.at[pl.ds(goff, GSZ), :], ebuf)
        pltpu.async_copy(h_hbm.at[sidx], hbuf, sem).wait()

        def _row(r, _):
            for cc in range(H // 16):
                sl = pl.ds(cc * 16, 16)
                ebuf[r, sl] = jnp.maximum(hbuf[r, sl] + ebuf[r, sl], 0.0)
            return 0
        lax.fori_loop(0, GSZ, _row, 0)

        pltpu.sync_copy(ebuf, aggr.at[didx], add=True)
        return 0
    lax.fori_loop(0, ng, _group, 0)
    plsc.subcore_barrier()

    for k in range((NCH + NS - 1) // NS):
        ch = s + k * NS

        @pl.when(ch < NCH)
        def _():
            sl = pl.ds(ch * CH, CH)
            pltpu.sync_copy(aggr.at[sl, :], out_hbm.at[c, sl, :])


_mp_layer = functools.partial(
    pl.kernel,
    out_type=jax.ShapeDtypeStruct((NC, N, H), _f32),
    mesh=plsc.VectorSubcoreMesh(core_axis_name="c", subcore_axis_name="s"),
    scratch_types=[
        pltpu.VMEM((GSZ,), jnp.int32),
        pltpu.VMEM((GSZ,), jnp.int32),
        pltpu.VMEM((GSZ, H), _f32),
        pltpu.VMEM((GSZ, H), _f32),
        pltpu.VMEM((CH, H), _f32),
        pltpu.VMEM_SHARED((N, H), _f32),
        pltpu.SemaphoreType.DMA,
    ],
    compiler_params=pltpu.CompilerParams(use_tc_tiling_on_sc=False),
)(_mp_body)


def _node_embed_body(x_ref, w_ref, b_ref, o_ref):
    o_ref[...] = (
        jnp.dot(x_ref[...], w_ref[...], preferred_element_type=_f32)
        + b_ref[...]
    )


def _edge_mlp_body(a_ref, w1_ref, b1_ref, w2_ref, b2_ref, o_ref):
    t = jnp.maximum(
        jnp.dot(a_ref[...], w1_ref[...], preferred_element_type=_f32)
        + b1_ref[...], 0.0)
    o_ref[...] = (
        jnp.dot(t, w2_ref[...], preferred_element_type=_f32) + b2_ref[...]
    )


def _layer_dense_body(h_ref, p_ref, eps_ref, w1_ref, b1_ref, w2_ref, b2_ref,
                      gam_ref, bet_ref, o_ref):
    z = h_ref[...] * (1.0 + eps_ref[0, 0]) + p_ref[0] + p_ref[1]
    t = jnp.maximum(
        jnp.dot(z, w1_ref[...], preferred_element_type=_f32) + b1_ref[...],
        0.0)
    t = jnp.dot(t, w2_ref[...], preferred_element_type=_f32) + b2_ref[...]
    mu = jnp.mean(t, axis=0, keepdims=True)
    va = jnp.mean((t - mu) * (t - mu), axis=0, keepdims=True)
    o_ref[...] = jnp.maximum(
        (t - mu) * lax.rsqrt(va + 1e-5) * gam_ref[...] + bet_ref[...], 0.0)


def _pool_body(h_ref, b_ref, wp1_ref, bp1_ref, wp2_ref, bp2_ref,
               g_ref, z_ref):
    onehot = (b_ref[...] == lax.broadcasted_iota(jnp.int32, (1, G), 1)
              ).astype(_f32)
    sums = lax.dot_general(onehot, h_ref[...],
                           (((0,), (0,)), ((), ())),
                           preferred_element_type=_f32)
    counts = lax.dot_general(onehot, jnp.ones((N, 1), _f32),
                             (((0,), (0,)), ((), ())),
                             preferred_element_type=_f32)
    mean = sums / jnp.maximum(counts, 1.0)
    gcat = jnp.concatenate([mean, sums], axis=1)
    g_ref[...] = gcat
    t = jnp.maximum(
        jnp.dot(gcat, wp1_ref[...], preferred_element_type=_f32)
        + bp1_ref[...], 0.0)
    z_ref[...] = (
        jnp.dot(t, wp2_ref[...], preferred_element_type=_f32) + bp2_ref[...]
    )


def _full(shape):
    return pl.BlockSpec(shape, lambda *a: tuple(0 for _ in shape))


def kernel(x, edge_index, edge_attr, batch, W_node, b_node, W_e1, b_e1,
           W_e2, b_e2, convs, bns, W_p1, b_p1, W_p2, b_p2):
    src = edge_index[0]
    dst = edge_index[1]

    h = pl.pallas_call(
        _node_embed_body,
        out_shape=jax.ShapeDtypeStruct((N, H), _f32),
    )(x, W_node, b_node.reshape(1, H))

    EB = 8000
    e = pl.pallas_call(
        _edge_mlp_body,
        grid=(E // EB,),
        in_specs=[
            pl.BlockSpec((EB, DE), lambda i: (i, 0)),
            _full((DE, H)), _full((1, H)), _full((H, H)), _full((1, H)),
        ],
        out_specs=pl.BlockSpec((EB, H), lambda i: (i, 0)),
        out_shape=jax.ShapeDtypeStruct((E, H), _f32),
    )(edge_attr, W_e1, b_e1.reshape(1, H), W_e2, b_e2.reshape(1, H))

    for conv, bn in zip(convs, bns):
        parts = _mp_layer(h, e, src, dst)
        h = pl.pallas_call(
            _layer_dense_body,
            out_shape=jax.ShapeDtypeStruct((N, H), _f32),
        )(h, parts, conv["eps"].reshape(1, 1),
          conv["W1"], conv["b1"].reshape(1, H),
          conv["W2"], conv["b2"].reshape(1, H),
          bn["gamma"].reshape(1, H), bn["beta"].reshape(1, H))

    g, z = pl.pallas_call(
        _pool_body,
        out_shape=(jax.ShapeDtypeStruct((G, 2 * H), _f32),
                   jax.ShapeDtypeStruct((G, P), _f32)),
    )(h, batch.reshape(N, 1), W_p1, b_p1.reshape(1, H),
      W_p2, b_p2.reshape(1, P))
    return (g, z)


# speedup vs baseline: 4.5737x; 1.2862x over previous
import functools

import jax
import jax.numpy as jnp
from jax import lax
from jax.experimental import pallas as pl
from jax.experimental.pallas import tpu as pltpu
from jax.experimental.pallas import tpu_sc as plsc

N = 10000
E = 320000
D = 128
DE = 16
H = 64
P = 64
G = 32

NC = 2
NS = 16
NW = NC * NS
EPT = E // NW
GSZ = 100
NGPC = 4
CSZ = GSZ * NGPC
NSC = EPT // CSZ
CH = 200
NCH = N // CH

_f32 = jnp.float32


def _mp_body(h_hbm, e_hbm, src_hbm, dst_hbm, out_hbm,
             sidx, didx, hbuf, ebuf, aggr,
             sem_a0, sem_a1, sem_g, sem_s):
    c = lax.axis_index("c")
    s = lax.axis_index("s")
    wid = s * NC + c

    def _zr(r, _):
        for cc in range(H // 16):
            ebuf[0, r, pl.ds(cc * 16, 16)] = jnp.zeros((16,), _f32)
        return 0
    lax.fori_loop(0, CH, _zr, 0)
    for k in range((NCH + NS - 1) // NS):
        ch = s + k * NS

        @pl.when(ch < NCH)
        def _():
            pltpu.sync_copy(ebuf.at[0, pl.ds(0, CH), :],
                            aggr.at[pl.ds(ch * CH, CH), :])
    plsc.subcore_barrier()

    ebase = wid * EPT
    ibase = ebase // GSZ

    def _issue_loads(sc, b):
        irow = ibase + sc * NGPC
        eoff = ebase + sc * CSZ
        pltpu.async_copy(src_hbm.at[pl.ds(irow, NGPC), :], sidx.at[b],
                         sem_a0 if b == 0 else sem_a1)
        pltpu.async_copy(dst_hbm.at[pl.ds(irow, NGPC), :], didx.at[b],
                         sem_a0 if b == 0 else sem_a1)
        pltpu.async_copy(e_hbm.at[pl.ds(eoff, CSZ), :], ebuf.at[b],
                         sem_a0 if b == 0 else sem_a1)

    def _wait_loads(sc, b):
        irow = ibase + sc * NGPC
        eoff = ebase + sc * CSZ
        sem = sem_a0 if b == 0 else sem_a1
        pltpu.make_async_copy(src_hbm.at[pl.ds(irow, NGPC), :], sidx.at[b],
                              sem).wait()
        pltpu.make_async_copy(dst_hbm.at[pl.ds(irow, NGPC), :], didx.at[b],
                              sem).wait()
        pltpu.make_async_copy(e_hbm.at[pl.ds(eoff, CSZ), :], ebuf.at[b],
                              sem).wait()

    _issue_loads(0, 0)

    def _body(sc, b):
        b1 = 1 - b
        _wait_loads(sc, b)

        @pl.when(sc + 1 < NSC)
        def _():
            _issue_loads(sc + 1, b1)

        g0 = pltpu.async_copy(h_hbm.at[sidx.at[b, 0]],
                              hbuf.at[pl.ds(0, GSZ)], sem_g)
        g1 = pltpu.async_copy(h_hbm.at[sidx.at[b, 1]],
                              hbuf.at[pl.ds(GSZ, GSZ)], sem_g)
        g0.wait()
        g1.wait()
        g2 = pltpu.async_copy(h_hbm.at[sidx.at[b, 2]],
                              hbuf.at[pl.ds(2 * GSZ, GSZ)], sem_g)
        g3 = pltpu.async_copy(h_hbm.at[sidx.at[b, 3]],
                              hbuf.at[pl.ds(3 * GSZ, GSZ)], sem_g)

        def _row(r, _):
            for cc in range(H // 16):
                sl = pl.ds(cc * 16, 16)
                ebuf[b, r, sl] = jnp.maximum(
                    hbuf[r, sl] + ebuf[b, r, sl], 0.0)
            return 0
        lax.fori_loop(0, 2 * GSZ, _row, 0, unroll=2)
        s0 = pltpu.async_copy(ebuf.at[b, pl.ds(0, GSZ)],
                              aggr.at[didx.at[b, 0]], sem_s, add=True)
        s1 = pltpu.async_copy(ebuf.at[b, pl.ds(GSZ, GSZ)],
                              aggr.at[didx.at[b, 1]], sem_s, add=True)
        g2.wait()
        g3.wait()
        lax.fori_loop(2 * GSZ, 4 * GSZ, _row, 0, unroll=2)
        s2 = pltpu.async_copy(ebuf.at[b, pl.ds(2 * GSZ, GSZ)],
                              aggr.at[didx.at[b, 2]], sem_s, add=True)
        s3 = pltpu.async_copy(ebuf.at[b, pl.ds(3 * GSZ, GSZ)],
                              aggr.at[didx.at[b, 3]], sem_s, add=True)
        s0.wait()
        s1.wait()
        s2.wait()
        s3.wait()

    def _outer(o, _):
        for b in range(2):
            sc = o * 2 + b

            @pl.when(sc < NSC)
            def _():
                _body(sc, b)
        return 0
    lax.fori_loop(0, (NSC + 1) // 2, _outer, 0)
    plsc.subcore_barrier()

    for k in range((NCH + NS - 1) // NS):
        ch = s + k * NS

        @pl.when(ch < NCH)
        def _():
            sl = pl.ds(ch * CH, CH)
            pltpu.sync_copy(aggr.at[sl, :], out_hbm.at[c, sl, :])


_mp_layer = functools.partial(
    pl.kernel,
    out_type=jax.ShapeDtypeStruct((NC, N, H), _f32),
    mesh=plsc.VectorSubcoreMesh(core_axis_name="c", subcore_axis_name="s"),
    scratch_types=[
        pltpu.VMEM((2, NGPC, GSZ), jnp.int32),
        pltpu.VMEM((2, NGPC, GSZ), jnp.int32),
        pltpu.VMEM((CSZ, H), _f32),
        pltpu.VMEM((2, CSZ, H), _f32),
        pltpu.VMEM_SHARED((N, H), _f32),
        pltpu.SemaphoreType.DMA,
        pltpu.SemaphoreType.DMA,
        pltpu.SemaphoreType.DMA,
        pltpu.SemaphoreType.DMA,
    ],
    compiler_params=pltpu.CompilerParams(use_tc_tiling_on_sc=False),
)(_mp_body)


def _node_embed_body(x_ref, w_ref, b_ref, o_ref):
    o_ref[...] = (
        jnp.dot(x_ref[...], w_ref[...], preferred_element_type=_f32)
        + b_ref[...]
    )


def _edge_mlp_body(a_ref, w1_ref, b1_ref, w2_ref, b2_ref, o_ref):
    t = jnp.maximum(
        jnp.dot(a_ref[...], w1_ref[...], preferred_element_type=_f32)
        + b1_ref[...], 0.0)
    o_ref[...] = (
        jnp.dot(t, w2_ref[...], preferred_element_type=_f32) + b2_ref[...]
    )


def _layer_dense_body(h_ref, p_ref, eps_ref, w1_ref, b1_ref, w2_ref, b2_ref,
                      gam_ref, bet_ref, o_ref):
    z = h_ref[...] * (1.0 + eps_ref[0, 0]) + p_ref[0] + p_ref[1]
    t = jnp.maximum(
        jnp.dot(z, w1_ref[...], preferred_element_type=_f32) + b1_ref[...],
        0.0)
    t = jnp.dot(t, w2_ref[...], preferred_element_type=_f32) + b2_ref[...]
    mu = jnp.mean(t, axis=0, keepdims=True)
    va = jnp.mean((t - mu) * (t - mu), axis=0, keepdims=True)
    o_ref[...] = jnp.maximum(
        (t - mu) * lax.rsqrt(va + 1e-5) * gam_ref[...] + bet_ref[...], 0.0)


def _pool_body(h_ref, b_ref, wp1_ref, bp1_ref, wp2_ref, bp2_ref,
               g_ref, z_ref):
    onehot = (b_ref[...] == lax.broadcasted_iota(jnp.int32, (1, G), 1)
              ).astype(_f32)
    sums = lax.dot_general(onehot, h_ref[...],
                           (((0,), (0,)), ((), ())),
                           preferred_element_type=_f32)
    counts = lax.dot_general(onehot, jnp.ones((N, 1), _f32),
                             (((0,), (0,)), ((), ())),
                             preferred_element_type=_f32)
    mean = sums / jnp.maximum(counts, 1.0)
    gcat = jnp.concatenate([mean, sums], axis=1)
    g_ref[...] = gcat
    t = jnp.maximum(
        jnp.dot(gcat, wp1_ref[...], preferred_element_type=_f32)
        + bp1_ref[...], 0.0)
    z_ref[...] = (
        jnp.dot(t, wp2_ref[...], preferred_element_type=_f32) + bp2_ref[...]
    )


def _full(shape):
    return pl.BlockSpec(shape, lambda *a: tuple(0 for _ in shape))


def kernel(x, edge_index, edge_attr, batch, W_node, b_node, W_e1, b_e1,
           W_e2, b_e2, convs, bns, W_p1, b_p1, W_p2, b_p2):
    src = edge_index[0].reshape(E // GSZ, GSZ)
    dst = edge_index[1].reshape(E // GSZ, GSZ)

    h = pl.pallas_call(
        _node_embed_body,
        out_shape=jax.ShapeDtypeStruct((N, H), _f32),
    )(x, W_node, b_node.reshape(1, H))

    EB = 8000
    e = pl.pallas_call(
        _edge_mlp_body,
        grid=(E // EB,),
        in_specs=[
            pl.BlockSpec((EB, DE), lambda i: (i, 0)),
            _full((DE, H)), _full((1, H)), _full((H, H)), _full((1, H)),
        ],
        out_specs=pl.BlockSpec((EB, H), lambda i: (i, 0)),
        out_shape=jax.ShapeDtypeStruct((E, H), _f32),
    )(edge_attr, W_e1, b_e1.reshape(1, H), W_e2, b_e2.reshape(1, H))

    for conv, bn in zip(convs, bns):
        parts = _mp_layer(h, e, src, dst)
        h = pl.pallas_call(
            _layer_dense_body,
            out_shape=jax.ShapeDtypeStruct((N, H), _f32),
        )(h, parts, conv["eps"].reshape(1, 1),
          conv["W1"], conv["b1"].reshape(1, H),
          conv["W2"], conv["b2"].reshape(1, H),
          bn["gamma"].reshape(1, H), bn["beta"].reshape(1, H))

    g, z = pl.pallas_call(
        _pool_body,
        out_shape=(jax.ShapeDtypeStruct((G, 2 * H), _f32),
                   jax.ShapeDtypeStruct((G, P), _f32)),
    )(h, batch.reshape(N, 1), W_p1, b_p1.reshape(1, H),
      W_p2, b_p2.reshape(1, P))
    return (g, z)


# speedup vs baseline: 5.1538x; 1.1268x over previous
import functools

import jax
import jax.numpy as jnp
from jax import lax
from jax.experimental import pallas as pl
from jax.experimental.pallas import tpu as pltpu
from jax.experimental.pallas import tpu_sc as plsc

N = 10000
E = 320000
D = 128
DE = 16
H = 64
P = 64
G = 32

NC = 2
NS = 16
NW = NC * NS
EPT = E // NW
GSZ = 100
NGPC = 4
CSZ = GSZ * NGPC
NSC = EPT // CSZ
CH = 200
NCH = N // CH

_f32 = jnp.float32


def _mp_body(h_hbm, e_hbm, src_hbm, dst_hbm, out_hbm,
             sidx, didx, hbuf, ebuf, aggr,
             sem_a0, sem_a1, sem_g, sem_s):
    c = lax.axis_index("c")
    s = lax.axis_index("s")
    wid = s * NC + c

    def _zr(r, _):
        for cc in range(H // 16):
            hbuf[r, pl.ds(cc * 16, 16)] = jnp.zeros((16,), _f32)
        return 0
    lax.fori_loop(0, CH, _zr, 0)
    for k in range((NCH + NS - 1) // NS):
        ch = s + k * NS

        @pl.when(ch < NCH)
        def _():
            pltpu.sync_copy(hbuf.at[pl.ds(0, CH), :],
                            aggr.at[pl.ds(ch * CH, CH), :])
    plsc.subcore_barrier()

    ebase = wid * EPT
    ibase = ebase // GSZ

    def _issue_loads(sc, b):
        irow = ibase + sc * NGPC
        erow = (ebase + sc * CSZ) // 2
        sem = sem_a0 if b == 0 else sem_a1
        pltpu.async_copy(src_hbm.at[pl.ds(irow, NGPC), :], sidx.at[b], sem)
        pltpu.async_copy(dst_hbm.at[pl.ds(irow, NGPC), :], didx.at[b], sem)
        pltpu.async_copy(e_hbm.at[pl.ds(erow, CSZ // 2), :], ebuf.at[b], sem)

    def _wait_loads(sc, b):
        irow = ibase + sc * NGPC
        erow = (ebase + sc * CSZ) // 2
        sem = sem_a0 if b == 0 else sem_a1
        pltpu.make_async_copy(src_hbm.at[pl.ds(irow, NGPC), :], sidx.at[b],
                              sem).wait()
        pltpu.make_async_copy(dst_hbm.at[pl.ds(irow, NGPC), :], didx.at[b],
                              sem).wait()
        pltpu.make_async_copy(e_hbm.at[pl.ds(erow, CSZ // 2), :], ebuf.at[b],
                              sem).wait()

    _issue_loads(0, 0)

    def _body(sc, b):
        b1 = 1 - b
        _wait_loads(sc, b)

        @pl.when(sc + 1 < NSC)
        def _():
            _issue_loads(sc + 1, b1)

        g0 = pltpu.async_copy(h_hbm.at[sidx.at[b, 0]],
                              hbuf.at[pl.ds(0, GSZ)], sem_g)
        g1 = pltpu.async_copy(h_hbm.at[sidx.at[b, 1]],
                              hbuf.at[pl.ds(GSZ, GSZ)], sem_g)
        g0.wait()
        g1.wait()
        g2 = pltpu.async_copy(h_hbm.at[sidx.at[b, 2]],
                              hbuf.at[pl.ds(2 * GSZ, GSZ)], sem_g)
        g3 = pltpu.async_copy(h_hbm.at[sidx.at[b, 3]],
                              hbuf.at[pl.ds(3 * GSZ, GSZ)], sem_g)

        def _row(r2, _):
            for half in range(2):
                r = 2 * r2 + half
                for cc in range(H // 16):
                    sl = pl.ds(cc * 16, 16)
                    el = pl.ds(half * H + cc * 16, 16)
                    hbuf[r, sl] = jnp.maximum(
                        hbuf[r, sl] + ebuf[b, r2, el], 0.0)
            return 0
        lax.fori_loop(0, GSZ, _row, 0, unroll=2)
        s0 = pltpu.async_copy(hbuf.at[pl.ds(0, GSZ)],
                              aggr.at[didx.at[b, 0]], sem_s, add=True)
        s1 = pltpu.async_copy(hbuf.at[pl.ds(GSZ, GSZ)],
                              aggr.at[didx.at[b, 1]], sem_s, add=True)
        g2.wait()
        g3.wait()
        lax.fori_loop(GSZ, 2 * GSZ, _row, 0, unroll=2)
        s2 = pltpu.async_copy(hbuf.at[pl.ds(2 * GSZ, GSZ)],
                              aggr.at[didx.at[b, 2]], sem_s, add=True)
        s3 = pltpu.async_copy(hbuf.at[pl.ds(3 * GSZ, GSZ)],
                              aggr.at[didx.at[b, 3]], sem_s, add=True)
        s0.wait()
        s1.wait()
        s2.wait()
        s3.wait()

    def _outer(o, _):
        for b in range(2):
            sc = o * 2 + b

            @pl.when(sc < NSC)
            def _():
                _body(sc, b)
        return 0
    lax.fori_loop(0, (NSC + 1) // 2, _outer, 0)
    plsc.subcore_barrier()

    for k in range((NCH + NS - 1) // NS):
        ch = s + k * NS

        @pl.when(ch < NCH)
        def _():
            sl = pl.ds(ch * CH, CH)
            pltpu.sync_copy(aggr.at[sl, :], out_hbm.at[c, sl, :])


_mp_layer = functools.partial(
    pl.kernel,
    out_type=jax.ShapeDtypeStruct((NC, N, H), _f32),
    mesh=plsc.VectorSubcoreMesh(core_axis_name="c", subcore_axis_name="s"),
    scratch_types=[
        pltpu.VMEM((2, NGPC, GSZ), jnp.int32),
        pltpu.VMEM((2, NGPC, GSZ), jnp.int32),
        pltpu.VMEM((CSZ, H), _f32),
        pltpu.VMEM((2, CSZ // 2, 2 * H), _f32),
        pltpu.VMEM_SHARED((N, H), _f32),
        pltpu.SemaphoreType.DMA,
        pltpu.SemaphoreType.DMA,
        pltpu.SemaphoreType.DMA,
        pltpu.SemaphoreType.DMA,
    ],
    compiler_params=pltpu.CompilerParams(use_tc_tiling_on_sc=False),
)(_mp_body)


def _node_embed_body(x_ref, w_ref, b_ref, o_ref):
    o_ref[...] = (
        jnp.dot(x_ref[...], w_ref[...], preferred_element_type=_f32)
        + b_ref[...]
    )


def _edge_mlp_body(a_ref, w1_ref, b1_ref, w2_ref, b2_ref, o_ref):
    w1 = w1_ref[...]
    z1 = jnp.zeros((DE, H), _f32)
    w1b = jnp.concatenate(
        [jnp.concatenate([w1, z1], axis=1),
         jnp.concatenate([z1, w1], axis=1)], axis=0)
    w2 = w2_ref[...]
    z2 = jnp.zeros((H, H), _f32)
    w2b = jnp.concatenate(
        [jnp.concatenate([w2, z2], axis=1),
         jnp.concatenate([z2, w2], axis=1)], axis=0)
    b1b = jnp.concatenate([b1_ref[...], b1_ref[...]], axis=1)
    b2b = jnp.concatenate([b2_ref[...], b2_ref[...]], axis=1)
    t = jnp.maximum(
        jnp.dot(a_ref[...], w1b, preferred_element_type=_f32) + b1b, 0.0)
    o_ref[...] = jnp.dot(t, w2b, preferred_element_type=_f32) + b2b


def _layer_dense_body(h_ref, p_ref, eps_ref, w1_ref, b1_ref, w2_ref, b2_ref,
                      gam_ref, bet_ref, o_ref):
    z = h_ref[...] * (1.0 + eps_ref[0, 0]) + p_ref[0] + p_ref[1]
    t = jnp.maximum(
        jnp.dot(z, w1_ref[...], preferred_element_type=_f32) + b1_ref[...],
        0.0)
    t = jnp.dot(t, w2_ref[...], preferred_element_type=_f32) + b2_ref[...]
    mu = jnp.mean(t, axis=0, keepdims=True)
    va = jnp.mean((t - mu) * (t - mu), axis=0, keepdims=True)
    o_ref[...] = jnp.maximum(
        (t - mu) * lax.rsqrt(va + 1e-5) * gam_ref[...] + bet_ref[...], 0.0)


def _pool_body(h_ref, b_ref, wp1_ref, bp1_ref, wp2_ref, bp2_ref,
               g_ref, z_ref):
    onehot = (b_ref[...] == lax.broadcasted_iota(jnp.int32, (1, G), 1)
              ).astype(_f32)
    sums = lax.dot_general(onehot, h_ref[...],
                           (((0,), (0,)), ((), ())),
                           preferred_element_type=_f32)
    counts = lax.dot_general(onehot, jnp.ones((N, 1), _f32),
                             (((0,), (0,)), ((), ())),
                             preferred_element_type=_f32)
    mean = sums / jnp.maximum(counts, 1.0)
    gcat = jnp.concatenate([mean, sums], axis=1)
    g_ref[...] = gcat
    t = jnp.maximum(
        jnp.dot(gcat, wp1_ref[...], preferred_element_type=_f32)
        + bp1_ref[...], 0.0)
    z_ref[...] = (
        jnp.dot(t, wp2_ref[...], preferred_element_type=_f32) + bp2_ref[...]
    )


def _full(shape):
    return pl.BlockSpec(shape, lambda *a: tuple(0 for _ in shape))


def kernel(x, edge_index, edge_attr, batch, W_node, b_node, W_e1, b_e1,
           W_e2, b_e2, convs, bns, W_p1, b_p1, W_p2, b_p2):
    src = edge_index[0].reshape(E // GSZ, GSZ)
    dst = edge_index[1].reshape(E // GSZ, GSZ)

    h = pl.pallas_call(
        _node_embed_body,
        out_shape=jax.ShapeDtypeStruct((N, H), _f32),
    )(x, W_node, b_node.reshape(1, H))

    EB = 4000
    e = pl.pallas_call(
        _edge_mlp_body,
        grid=(E // 2 // EB,),
        in_specs=[
            pl.BlockSpec((EB, 2 * DE), lambda i: (i, 0)),
            _full((DE, H)), _full((1, H)), _full((H, H)), _full((1, H)),
        ],
        out_specs=pl.BlockSpec((EB, 2 * H), lambda i: (i, 0)),
        out_shape=jax.ShapeDtypeStruct((E // 2, 2 * H), _f32),
    )(edge_attr.reshape(E // 2, 2 * DE), W_e1, b_e1.reshape(1, H),
      W_e2, b_e2.reshape(1, H))

    for conv, bn in zip(convs, bns):
        parts = _mp_layer(h, e, src, dst)
        h = pl.pallas_call(
            _layer_dense_body,
            out_shape=jax.ShapeDtypeStruct((N, H), _f32),
        )(h, parts, conv["eps"].reshape(1, 1),
          conv["W1"], conv["b1"].reshape(1, H),
          conv["W2"], conv["b2"].reshape(1, H),
          bn["gamma"].reshape(1, H), bn["beta"].reshape(1, H))

    g, z = pl.pallas_call(
        _pool_body,
        out_shape=(jax.ShapeDtypeStruct((G, 2 * H), _f32),
                   jax.ShapeDtypeStruct((G, P), _f32)),
    )(h, batch.reshape(N, 1), W_p1, b_p1.reshape(1, H),
      W_p2, b_p2.reshape(1, P))
    return (g, z)


# speedup vs baseline: 5.3790x; 1.0437x over previous
import functools

import jax
import jax.numpy as jnp
from jax import lax
from jax.experimental import pallas as pl
from jax.experimental.pallas import tpu as pltpu
from jax.experimental.pallas import tpu_sc as plsc

N = 10000
E = 320000
D = 128
DE = 16
H = 64
P = 64
G = 32

NC = 2
NS = 16
NW = NC * NS
EPT = E // NW
GSZ = 100
NGPC = 4
CSZ = GSZ * NGPC
NSC = EPT // CSZ
CH = 200
NCH = N // CH

_f32 = jnp.float32


def _mp_body(h_hbm, e_hbm, src_hbm, dst_hbm, out_hbm,
             sidx, didx, hbuf, ebuf, aggr,
             sem_a0, sem_a1, sem_g, sem_s):
    c = lax.axis_index("c")
    s = lax.axis_index("s")
    wid = s * NC + c

    def _zr(r, _):
        for cc in range(H // 16):
            hbuf[r, pl.ds(cc * 16, 16)] = jnp.zeros((16,), _f32)
        return 0
    lax.fori_loop(0, CH, _zr, 0)
    for k in range((NCH + NS - 1) // NS):
        ch = s + k * NS

        @pl.when(ch < NCH)
        def _():
            pltpu.sync_copy(hbuf.at[pl.ds(0, CH), :],
                            aggr.at[pl.ds(ch * CH, CH), :])
    plsc.subcore_barrier()

    ebase = wid * EPT
    ibase = ebase // GSZ

    def _issue_loads(sc, b):
        irow = ibase + sc * NGPC
        erow = (ebase + sc * CSZ) // 2
        sem = sem_a0 if b == 0 else sem_a1
        pltpu.async_copy(src_hbm.at[pl.ds(irow, NGPC), :], sidx.at[b], sem)
        pltpu.async_copy(dst_hbm.at[pl.ds(irow, NGPC), :], didx.at[b], sem)
        pltpu.async_copy(e_hbm.at[pl.ds(erow, CSZ // 2), :], ebuf.at[b], sem)

    def _wait_loads(sc, b):
        irow = ibase + sc * NGPC
        erow = (ebase + sc * CSZ) // 2
        sem = sem_a0 if b == 0 else sem_a1
        pltpu.make_async_copy(src_hbm.at[pl.ds(irow, NGPC), :], sidx.at[b],
                              sem).wait()
        pltpu.make_async_copy(dst_hbm.at[pl.ds(irow, NGPC), :], didx.at[b],
                              sem).wait()
        pltpu.make_async_copy(e_hbm.at[pl.ds(erow, CSZ // 2), :], ebuf.at[b],
                              sem).wait()

    def _gA(b):
        return (pltpu.async_copy(h_hbm.at[sidx.at[b, 0]],
                                 hbuf.at[pl.ds(0, GSZ)], sem_g),
                pltpu.async_copy(h_hbm.at[sidx.at[b, 1]],
                                 hbuf.at[pl.ds(GSZ, GSZ)], sem_g))

    def _gB(b):
        return (pltpu.async_copy(h_hbm.at[sidx.at[b, 2]],
                                 hbuf.at[pl.ds(2 * GSZ, GSZ)], sem_g),
                pltpu.async_copy(h_hbm.at[sidx.at[b, 3]],
                                 hbuf.at[pl.ds(3 * GSZ, GSZ)], sem_g))

    def _sA(b):
        return (pltpu.async_copy(hbuf.at[pl.ds(0, GSZ)],
                                 aggr.at[didx.at[b, 0]], sem_s, add=True),
                pltpu.async_copy(hbuf.at[pl.ds(GSZ, GSZ)],
                                 aggr.at[didx.at[b, 1]], sem_s, add=True))

    def _sB(b):
        return (pltpu.async_copy(hbuf.at[pl.ds(2 * GSZ, GSZ)],
                                 aggr.at[didx.at[b, 2]], sem_s, add=True),
                pltpu.async_copy(hbuf.at[pl.ds(3 * GSZ, GSZ)],
                                 aggr.at[didx.at[b, 3]], sem_s, add=True))

    def _wait(ds_):
        for d in ds_:
            d.wait()

    def _compute(b, lo, hi):
        def _row(r2, _):
            for half in range(2):
                r = 2 * r2 + half
                for cc in range(H // 16):
                    sl = pl.ds(cc * 16, 16)
                    el = pl.ds(half * H + cc * 16, 16)
                    hbuf[r, sl] = jnp.maximum(
                        hbuf[r, sl] + ebuf[b, r2, el], 0.0)
            return 0
        lax.fori_loop(lo, hi, _row, 0, unroll=4)

    def _wait_sidx(sc, b):
        irow = ibase + sc * NGPC
        sem = sem_a0 if b == 0 else sem_a1
        pltpu.make_async_copy(src_hbm.at[pl.ds(irow, NGPC), :], sidx.at[b],
                              sem).wait()

    def _wait_de(sc, b):
        irow = ibase + sc * NGPC
        erow = (ebase + sc * CSZ) // 2
        sem = sem_a0 if b == 0 else sem_a1
        pltpu.make_async_copy(dst_hbm.at[pl.ds(irow, NGPC), :], didx.at[b],
                              sem).wait()
        pltpu.make_async_copy(e_hbm.at[pl.ds(erow, CSZ // 2), :], ebuf.at[b],
                              sem).wait()

    def _issue_next_loads(sc, b):
        @pl.when(sc + 1 < NSC)
        def _():
            _issue_loads(sc + 1, 1 - b)

    _issue_loads(0, 0)

    def _group(o, base, nk):
        ga = None
        sb_prev = None
        for k in range(nk):
            sc = (base + k) if o is None else (o * 4 + k)
            b = k % 2
            if ga is None:
                _wait_loads(sc, b)
            else:
                _wait_de(sc, b)
            if sb_prev is not None:
                _wait(sb_prev)
            _issue_next_loads(sc, b)
            if ga is None:
                ga = _gA(b)
            gb = _gB(b)
            _wait(ga)
            ga = None
            _compute(b, 0, GSZ)
            sa = _sA(b)
            _wait(gb)
            _compute(b, GSZ, 2 * GSZ)
            sb_prev = _sB(b)
            _wait(sa)
            if k < nk - 1:
                _wait_sidx(sc + 1, 1 - b)
                ga = _gA(1 - b)
        _wait(sb_prev)

    def _outer(o, _):
        _group(o, None, 4)
        return 0
    lax.fori_loop(0, NSC // 4, _outer, 0)
    if NSC % 4:
        _group(None, (NSC // 4) * 4, NSC % 4)
    plsc.subcore_barrier()

    for k in range((NCH + NS - 1) // NS):
        ch = s + k * NS

        @pl.when(ch < NCH)
        def _():
            sl = pl.ds(ch * CH, CH)
            pltpu.sync_copy(aggr.at[sl, :], out_hbm.at[c, sl, :])


_mp_layer = functools.partial(
    pl.kernel,
    out_type=jax.ShapeDtypeStruct((NC, N, H), _f32),
    mesh=plsc.VectorSubcoreMesh(core_axis_name="c", subcore_axis_name="s"),
    scratch_types=[
        pltpu.VMEM((2, NGPC, GSZ), jnp.int32),
        pltpu.VMEM((2, NGPC, GSZ), jnp.int32),
        pltpu.VMEM((CSZ, H), _f32),
        pltpu.VMEM((2, CSZ // 2, 2 * H), _f32),
        pltpu.VMEM_SHARED((N, H), _f32),
        pltpu.SemaphoreType.DMA,
        pltpu.SemaphoreType.DMA,
        pltpu.SemaphoreType.DMA,
        pltpu.SemaphoreType.DMA,
    ],
    compiler_params=pltpu.CompilerParams(use_tc_tiling_on_sc=False),
)(_mp_body)


def _node_embed_body(x_ref, w_ref, b_ref, o_ref):
    o_ref[...] = (
        jnp.dot(x_ref[...], w_ref[...], preferred_element_type=_f32)
        + b_ref[...]
    )


def _edge_mlp_body(a_ref, w1_ref, b1_ref, w2_ref, b2_ref, o_ref):
    w1 = w1_ref[...]
    z1 = jnp.zeros((DE, H), _f32)
    w1b = jnp.concatenate(
        [jnp.concatenate([w1, z1], axis=1),
         jnp.concatenate([z1, w1], axis=1)], axis=0)
    w2 = w2_ref[...]
    z2 = jnp.zeros((H, H), _f32)
    w2b = jnp.concatenate(
        [jnp.concatenate([w2, z2], axis=1),
         jnp.concatenate([z2, w2], axis=1)], axis=0)
    b1b = jnp.concatenate([b1_ref[...], b1_ref[...]], axis=1)
    b2b = jnp.concatenate([b2_ref[...], b2_ref[...]], axis=1)
    t = jnp.maximum(
        jnp.dot(a_ref[...], w1b, preferred_element_type=_f32) + b1b, 0.0)
    o_ref[...] = jnp.dot(t, w2b, preferred_element_type=_f32) + b2b


def _layer_dense_body(h_ref, p_ref, eps_ref, w1_ref, b1_ref, w2_ref, b2_ref,
                      gam_ref, bet_ref, o_ref):
    z = h_ref[...] * (1.0 + eps_ref[0, 0]) + p_ref[0] + p_ref[1]
    t = jnp.maximum(
        jnp.dot(z, w1_ref[...], preferred_element_type=_f32) + b1_ref[...],
        0.0)
    t = jnp.dot(t, w2_ref[...], preferred_element_type=_f32) + b2_ref[...]
    mu = jnp.mean(t, axis=0, keepdims=True)
    va = jnp.mean((t - mu) * (t - mu), axis=0, keepdims=True)
    o_ref[...] = jnp.maximum(
        (t - mu) * lax.rsqrt(va + 1e-5) * gam_ref[...] + bet_ref[...], 0.0)


def _pool_body(h_ref, b_ref, wp1_ref, bp1_ref, wp2_ref, bp2_ref,
               g_ref, z_ref):
    onehot = (b_ref[...] == lax.broadcasted_iota(jnp.int32, (1, G), 1)
              ).astype(_f32)
    sums = lax.dot_general(onehot, h_ref[...],
                           (((0,), (0,)), ((), ())),
                           preferred_element_type=_f32)
    counts = lax.dot_general(onehot, jnp.ones((N, 1), _f32),
                             (((0,), (0,)), ((), ())),
                             preferred_element_type=_f32)
    mean = sums / jnp.maximum(counts, 1.0)
    gcat = jnp.concatenate([mean, sums], axis=1)
    g_ref[...] = gcat
    t = jnp.maximum(
        jnp.dot(gcat, wp1_ref[...], preferred_element_type=_f32)
        + bp1_ref[...], 0.0)
    z_ref[...] = (
        jnp.dot(t, wp2_ref[...], preferred_element_type=_f32) + bp2_ref[...]
    )


def _full(shape):
    return pl.BlockSpec(shape, lambda *a: tuple(0 for _ in shape))


def kernel(x, edge_index, edge_attr, batch, W_node, b_node, W_e1, b_e1,
           W_e2, b_e2, convs, bns, W_p1, b_p1, W_p2, b_p2):
    src = edge_index[0].reshape(E // GSZ, GSZ)
    dst = edge_index[1].reshape(E // GSZ, GSZ)

    h = pl.pallas_call(
        _node_embed_body,
        out_shape=jax.ShapeDtypeStruct((N, H), _f32),
    )(x, W_node, b_node.reshape(1, H))

    EB = 4000
    e = pl.pallas_call(
        _edge_mlp_body,
        grid=(E // 2 // EB,),
        in_specs=[
            pl.BlockSpec((EB, 2 * DE), lambda i: (i, 0)),
            _full((DE, H)), _full((1, H)), _full((H, H)), _full((1, H)),
        ],
        out_specs=pl.BlockSpec((EB, 2 * H), lambda i: (i, 0)),
        out_shape=jax.ShapeDtypeStruct((E // 2, 2 * H), _f32),
    )(edge_attr.reshape(E // 2, 2 * DE), W_e1, b_e1.reshape(1, H),
      W_e2, b_e2.reshape(1, H))

    for conv, bn in zip(convs, bns):
        parts = _mp_layer(h, e, src, dst)
        h = pl.pallas_call(
            _layer_dense_body,
            out_shape=jax.ShapeDtypeStruct((N, H), _f32),
        )(h, parts, conv["eps"].reshape(1, 1),
          conv["W1"], conv["b1"].reshape(1, H),
          conv["W2"], conv["b2"].reshape(1, H),
          bn["gamma"].reshape(1, H), bn["beta"].reshape(1, H))

    g, z = pl.pallas_call(
        _pool_body,
        out_shape=(jax.ShapeDtypeStruct((G, 2 * H), _f32),
                   jax.ShapeDtypeStruct((G, P), _f32)),
    )(h, batch.reshape(N, 1), W_p1, b_p1.reshape(1, H),
      W_p2, b_p2.reshape(1, P))
    return (g, z)
